# baseline (speedup 1.0000x reference)
import functools

import jax
import jax.numpy as jnp
from jax import lax
from jax.experimental import pallas as pl
from jax.experimental.pallas import tpu as pltpu

_CompilerParams = getattr(pltpu, "CompilerParams", None)
if _CompilerParams is None:
    _CompilerParams = pltpu.TPUCompilerParams

_ANY = getattr(pl, "ANY", None)
if _ANY is None:
    _ANY = pltpu.MemorySpace.HBM

B_BLK = 64
_UNROLL = 8


def _ncf_kernel(
    item_idx_ref,
    user_idx_ref,
    wi_ref,
    wug_ref,
    wum_ref,
    bug_ref, bum_ref,
    bi_ref,
    w1_ref, b1_ref, w2_ref, b2_ref, w3_ref, b3_ref, wp_ref, bp_ref,
    out_ref,
    scr_i,
    scr_ug, scr_um,
    idx_smem,
    uidx_smem,
    sem_si, sem_su, sem_i, sem_ug, sem_um,
    *, n_items: int,
):
    m_rows = B_BLK * n_items

    pltpu.make_async_copy(item_idx_ref, idx_smem, sem_si).start()
    pltpu.make_async_copy(user_idx_ref, uidx_smem, sem_su).start()
    pltpu.make_async_copy(item_idx_ref, idx_smem, sem_si).wait()

    def issue_chunk(c, _):
        base = c * _UNROLL
        for i in range(_UNROLL):
            k = base + i
            t = idx_smem[0, 0, k]
            pltpu.make_async_copy(wi_ref.at[t], scr_i.at[k, 0], sem_i).start(
                priority=i % 2)
        return ()
    lax.fori_loop(0, m_rows // _UNROLL, issue_chunk, ())

    pltpu.make_async_copy(user_idx_ref, uidx_smem, sem_su).wait()
    for u in range(B_BLK):
        t = uidx_smem[0, 0, u]
        pltpu.make_async_copy(wug_ref.at[t], scr_ug.at[u, 0], sem_ug).start()
        pltpu.make_async_copy(wum_ref.at[t], scr_um.at[u, 0], sem_um).start()

    pltpu.make_async_copy(scr_ug, scr_ug, sem_ug).wait()
    pltpu.make_async_copy(scr_um, scr_um, sem_um).wait()
    eu_g = scr_ug.reshape(B_BLK, 64)[...] + bug_ref[...]
    eu_m = scr_um.reshape(B_BLK, 64)[...] + bum_ref[...]

    w1 = w1_ref[...]
    u1 = jnp.dot(eu_m, w1[0:64, :], preferred_element_type=jnp.float32)

    k_io = lax.broadcasted_iota(jnp.int32, (m_rows, B_BLK), 0)
    u_io = lax.broadcasted_iota(jnp.int32, (m_rows, B_BLK), 1) * n_items
    r_mat = ((k_io >= u_io) & (k_io < u_io + n_items)).astype(jnp.float32)

    eu_g_rep = jnp.dot(r_mat, eu_g, preferred_element_type=jnp.float32)
    u1_rep = jnp.dot(r_mat, u1, preferred_element_type=jnp.float32)

    w1i_ext = jnp.concatenate([jnp.zeros((64, 128), jnp.float32), w1[64:128, :]], axis=0)

    pltpu.make_async_copy(scr_i, scr_i, sem_i).wait()
    full = scr_i.reshape(m_rows, 128)[...] + bi_ref[...]
    gmf = eu_g_rep * full[:, 0:64]

    i1 = jnp.dot(full, w1i_ext, preferred_element_type=jnp.float32)
    h1 = jnp.maximum(u1_rep + i1 + b1_ref[...], 0.0)
    h2 = jnp.maximum(
        jnp.dot(h1, w2_ref[...], preferred_element_type=jnp.float32) + b2_ref[...], 0.0)
    h3 = jnp.maximum(
        jnp.dot(h2, w3_ref[...], preferred_element_type=jnp.float32) + b3_ref[...], 0.0)

    wp = wp_ref[...]
    logit = (jnp.dot(gmf, wp[0:64, :], preferred_element_type=jnp.float32)
             + jnp.dot(h3, wp[64:96, :], preferred_element_type=jnp.float32)
             + bp_ref[...])
    out_ref[...] = jax.nn.sigmoid(logit)


def kernel(user, item, num_total, Wu_gmf, bu_gmf, Wu_mlp, bu_mlp,
           Wi_gmf, bi_gmf, Wi_mlp, bi_mlp, W1, b1, W2, b2, W3, b3, Wp, bp):
    batch, n_items = item.shape
    nb = batch // B_BLK
    nb2 = nb // 2
    m_rows = B_BLK * n_items
    embed = Wu_gmf.shape[1]

    item_idx = item.astype(jnp.int32).reshape(nb, 1, m_rows)
    user_idx = user.astype(jnp.int32).reshape(nb, 1, B_BLK)

    wi_cat = jnp.concatenate([Wi_gmf, Wi_mlp], axis=1)
    bi_cat = jnp.concatenate([bi_gmf, bi_mlp]).reshape(1, 2 * embed)
    biases = [b.reshape(1, -1) for b in (bu_gmf, bu_mlp, b1, b2, b3)]
    bp2 = bp.reshape(1, 1)

    in_specs = [
            pl.BlockSpec((1, 1, m_rows), lambda i: (i, 0, 0)),
            pl.BlockSpec((1, 1, B_BLK), lambda i: (i, 0, 0)),
            pl.BlockSpec(memory_space=_ANY),
            pl.BlockSpec(memory_space=_ANY),
            pl.BlockSpec(memory_space=_ANY),
            pl.BlockSpec((1, embed), lambda i: (0, 0)),
            pl.BlockSpec((1, embed), lambda i: (0, 0)),
            pl.BlockSpec((1, 2 * embed), lambda i: (0, 0)),
            pl.BlockSpec(W1.shape, lambda i: (0, 0)),
            pl.BlockSpec((1, 2 * embed), lambda i: (0, 0)),
            pl.BlockSpec(W2.shape, lambda i: (0, 0)),
            pl.BlockSpec((1, embed), lambda i: (0, 0)),
            pl.BlockSpec(W3.shape, lambda i: (0, 0)),
            pl.BlockSpec((1, embed // 2), lambda i: (0, 0)),
            pl.BlockSpec(Wp.shape, lambda i: (0, 0)),
            pl.BlockSpec((1, 1), lambda i: (0, 0)),
    ]

    pred = pl.pallas_call(
        functools.partial(_ncf_kernel, n_items=n_items),
        out_shape=jax.ShapeDtypeStruct((batch * n_items, 1), jnp.float32),
        grid=(nb,),
        in_specs=in_specs,
        out_specs=pl.BlockSpec((m_rows, 1), lambda i: (i, 0)),
        scratch_shapes=[
            pltpu.VMEM((m_rows, 1, 2 * embed), jnp.float32),
            pltpu.VMEM((B_BLK, 1, embed), jnp.float32),
            pltpu.VMEM((B_BLK, 1, embed), jnp.float32),
            pltpu.SMEM((1, 1, m_rows), jnp.int32),
            pltpu.SMEM((1, 1, B_BLK), jnp.int32),
            pltpu.SemaphoreType.DMA,
            pltpu.SemaphoreType.DMA,
            pltpu.SemaphoreType.DMA,
            pltpu.SemaphoreType.DMA,
            pltpu.SemaphoreType.DMA,
        ],
        compiler_params=_CompilerParams(
            dimension_semantics=("arbitrary",),
        ),
        name="ncf_fused",
    )(item_idx, user_idx, wi_cat, Wu_gmf, Wu_mlp, biases[0], biases[1],
      bi_cat, W1, biases[2], W2, biases[3], W3, biases[4], Wp, bp2)

    return pred.reshape(batch, n_items)
